# baseline (speedup 1.0000x reference)
import functools

import jax
import jax.numpy as jnp
from jax import lax
from jax.experimental import pallas as pl
from jax.experimental.pallas import tpu as pltpu
from jax.experimental.pallas import tpu_sc as plsc

ENT = 1000000
REL = 1000
H = 64
B = 4096
MARGIN = 1.0

NC = 2
NS = 16
NW = NC * NS
ROWS_PER_W = 4 * B // NW


def _sc_gather_body(idx_hbm, table_hbm, out_hbm, idx_v, rows_v, sem):
    wid = lax.axis_index("s") * NC + lax.axis_index("c")
    base = wid * ROWS_PER_W
    pltpu.sync_copy(idx_hbm.at[pl.ds(base, ROWS_PER_W)], idx_v)
    pltpu.async_copy(table_hbm.at[idx_v], rows_v, sem).wait()
    pltpu.sync_copy(rows_v, out_hbm.at[pl.ds(base, ROWS_PER_W)])


def _sc_gather(idx, table):
    mesh = plsc.VectorSubcoreMesh(core_axis_name="c", subcore_axis_name="s")
    return pl.kernel(
        _sc_gather_body,
        mesh=mesh,
        out_type=jax.ShapeDtypeStruct((4 * B, 2 * H), jnp.float32),
        scratch_types=[
            pltpu.VMEM((ROWS_PER_W,), jnp.int32),
            pltpu.VMEM((ROWS_PER_W, 2 * H), jnp.float32),
            pltpu.SemaphoreType.DMA,
        ],
    )(idx, table)


KU = 4


def _row64(rows2_ref, b, par):
    row = rows2_ref[pl.ds(b, 1), :]
    lo = lax.slice(row, (0, 0), (1, H))
    hi = lax.slice(row, (0, H), (1, 2 * H))
    return jnp.where(par > 0, hi, lo)


def _score_body(wp_ref, wn_ref, rows2_ref, rel_ref, out_ref, diff_ref):
    def body(i, acc):
        b0 = i * KU
        for u in range(KU):
            b = b0 + u
            wp = wp_ref[b]
            wn = wn_ref[b]
            rp = wp & 4095
            rn = wn & 4095
            Rp = rel_ref[pl.ds(rp * H, H), :]
            Rn = rel_ref[pl.ds(rn * H, H), :]
            hp = _row64(rows2_ref, b, (wp >> 12) & 1)
            tp = _row64(rows2_ref, B + b, (wp >> 13) & 1)
            hn = _row64(rows2_ref, 2 * B + b, (wn >> 12) & 1)
            tn = _row64(rows2_ref, 3 * B + b, (wn >> 13) & 1)
            pvec = jnp.dot(hp, Rp, preferred_element_type=jnp.float32) * tp
            nvec = jnp.dot(hn, Rn, preferred_element_type=jnp.float32) * tn
            diff_ref[pl.ds(b, 1), :] = nvec - pvec
        return acc

    lax.fori_loop(0, B // KU, body, jnp.float32(0.0))
    d = diff_ref[...]
    s = jnp.sum(d, axis=1) + MARGIN
    out_ref[0, 0] = jnp.sum(jnp.maximum(s, 0.0))


def _score(wp, wn, rows2, rel_v):
    return pl.pallas_call(
        _score_body,
        out_shape=jax.ShapeDtypeStruct((1, 1), jnp.float32),
        in_specs=[
            pl.BlockSpec(memory_space=pltpu.SMEM),
            pl.BlockSpec(memory_space=pltpu.SMEM),
            pl.BlockSpec(memory_space=pltpu.VMEM),
            pl.BlockSpec(memory_space=pltpu.VMEM),
        ],
        out_specs=pl.BlockSpec(memory_space=pltpu.SMEM),
        scratch_shapes=[
            pltpu.VMEM((B, H), jnp.float32),
        ],
    )(wp, wn, rows2, rel_v)


def kernel(pos_h, pos_t, pos_r, neg_h, neg_t, neg_r,
           ent_embeddings, rel_matrices):
    pos_h = pos_h.astype(jnp.int32)
    pos_t = pos_t.astype(jnp.int32)
    neg_h = neg_h.astype(jnp.int32)
    neg_t = neg_t.astype(jnp.int32)
    idx = jnp.concatenate([pos_h, pos_t, neg_h, neg_t])
    table2 = ent_embeddings.reshape(ENT // 2, 2 * H)
    rows2 = _sc_gather(idx >> 1, table2)
    wp = (pos_r.astype(jnp.int32) | ((pos_h & 1) << 12) | ((pos_t & 1) << 13))
    wn = (neg_r.astype(jnp.int32) | ((neg_h & 1) << 12) | ((neg_t & 1) << 13))
    rel_v = rel_matrices.reshape(REL * H, H)
    out = _score(wp, wn, rows2, rel_v)
    return out[0, 0]


# speedup vs baseline: 1.0743x; 1.0743x over previous
import functools

import jax
import jax.numpy as jnp
from jax import lax
from jax.experimental import pallas as pl
from jax.experimental.pallas import tpu as pltpu
from jax.experimental.pallas import tpu_sc as plsc

ENT = 1000000
REL = 1000
H = 64
B = 4096
MARGIN = 1.0

NC = 2
NS = 16
NW = NC * NS
ROWS_PER_W = 4 * B // NW


def _sc_gather_body(idx_hbm, table_hbm, out_hbm, idx_v, rows_v, sem):
    wid = lax.axis_index("s") * NC + lax.axis_index("c")
    base = wid * ROWS_PER_W
    pltpu.sync_copy(idx_hbm.at[pl.ds(base, ROWS_PER_W)], idx_v)
    pltpu.async_copy(table_hbm.at[idx_v], rows_v, sem).wait()
    pltpu.sync_copy(rows_v, out_hbm.at[pl.ds(base, ROWS_PER_W)])


def _sc_gather(idx, table):
    mesh = plsc.VectorSubcoreMesh(core_axis_name="c", subcore_axis_name="s")
    return pl.kernel(
        _sc_gather_body,
        mesh=mesh,
        out_type=jax.ShapeDtypeStruct((4 * B, 2 * H), jnp.float32),
        scratch_types=[
            pltpu.VMEM((ROWS_PER_W,), jnp.int32),
            pltpu.VMEM((ROWS_PER_W, 2 * H), jnp.float32),
            pltpu.SemaphoreType.DMA,
        ],
    )(idx, table)


KU = 8


def _blend(row, par_f):
    lo = lax.slice(row, (0, 0), (1, H))
    hi = lax.slice(row, (0, H), (1, 2 * H))
    return lo + (hi - lo) * par_f


def _deintl_matrix():
    ii = lax.broadcasted_iota(jnp.int32, (2 * H, 2 * H), 0)
    jj = lax.broadcasted_iota(jnp.int32, (2 * H, 2 * H), 1)
    src = (jnp.where(jj < 2 * H // 2, 0, 2 * H // 2)
           + 2 * (jj % (H // 2))
           + jnp.where((jj % H) < H // 2, 0, 1))
    return (ii == src).astype(jnp.float32)


def _score_body(wp_ref, wn_ref, rows2_ref, relp_ref, out_ref,
                hde_ref, diff_ref):
    P2 = _deintl_matrix()
    hde_ref[pl.ds(0, B), :] = jnp.dot(
        rows2_ref[pl.ds(0, B), :], P2, preferred_element_type=jnp.float32)
    hde_ref[pl.ds(B, B), :] = jnp.dot(
        rows2_ref[pl.ds(2 * B, B), :], P2, preferred_element_type=jnp.float32)

    def body(i, acc):
        b0 = i * KU
        for u in range(KU):
            b = b0 + u
            wp = wp_ref[b]
            wn = wn_ref[b]
            rp = wp & 4095
            rn = wn & 4095
            Rp = relp_ref[pl.ds(rp * 32, 32), :]
            Rn = relp_ref[pl.ds(rn * 32, 32), :]
            hpf = ((wp >> 12) & 1).astype(jnp.float32)
            tpf = ((wp >> 13) & 1).astype(jnp.float32)
            hnf = ((wn >> 12) & 1).astype(jnp.float32)
            tnf = ((wn >> 13) & 1).astype(jnp.float32)
            hp = _blend(hde_ref[pl.ds(b, 1), :], hpf)
            hn = _blend(hde_ref[pl.ds(B + b, 1), :], hnf)
            tp = _blend(rows2_ref[pl.ds(B + b, 1), :], tpf)
            tn = _blend(rows2_ref[pl.ds(3 * B + b, 1), :], tnf)
            pe = jnp.dot(lax.slice(hp, (0, 0), (1, 32)), Rp,
                         preferred_element_type=jnp.float32)
            po = jnp.dot(lax.slice(hp, (0, 32), (1, 64)), Rp,
                         preferred_element_type=jnp.float32)
            ne = jnp.dot(lax.slice(hn, (0, 0), (1, 32)), Rn,
                         preferred_element_type=jnp.float32)
            no = jnp.dot(lax.slice(hn, (0, 32), (1, 64)), Rn,
                         preferred_element_type=jnp.float32)
            pvec = (lax.slice(pe, (0, 0), (1, H))
                    + lax.slice(po, (0, H), (1, 2 * H))) * tp
            nvec = (lax.slice(ne, (0, 0), (1, H))
                    + lax.slice(no, (0, H), (1, 2 * H))) * tn
            diff_ref[pl.ds(b, 1), :] = nvec - pvec
        return acc

    lax.fori_loop(0, B // KU, body, jnp.float32(0.0))
    d = diff_ref[...]
    s = jnp.sum(d, axis=1) + MARGIN
    out_ref[0, 0] = jnp.sum(jnp.maximum(s, 0.0))


def _score(wp, wn, rows2, rel_pair):
    return pl.pallas_call(
        _score_body,
        out_shape=jax.ShapeDtypeStruct((1, 1), jnp.float32),
        in_specs=[
            pl.BlockSpec(memory_space=pltpu.SMEM),
            pl.BlockSpec(memory_space=pltpu.SMEM),
            pl.BlockSpec(memory_space=pltpu.VMEM),
            pl.BlockSpec(memory_space=pltpu.VMEM),
        ],
        out_specs=pl.BlockSpec(memory_space=pltpu.SMEM),
        scratch_shapes=[
            pltpu.VMEM((2 * B, 2 * H), jnp.float32),
            pltpu.VMEM((B, H), jnp.float32),
        ],
    )(wp, wn, rows2, rel_pair)


def kernel(pos_h, pos_t, pos_r, neg_h, neg_t, neg_r,
           ent_embeddings, rel_matrices):
    pos_h = pos_h.astype(jnp.int32)
    pos_t = pos_t.astype(jnp.int32)
    neg_h = neg_h.astype(jnp.int32)
    neg_t = neg_t.astype(jnp.int32)
    idx = jnp.concatenate([pos_h, pos_t, neg_h, neg_t])
    table2 = ent_embeddings.reshape(ENT // 2, 2 * H)
    rows2 = _sc_gather(idx >> 1, table2)
    wp = (pos_r.astype(jnp.int32) | ((pos_h & 1) << 12) | ((pos_t & 1) << 13))
    wn = (neg_r.astype(jnp.int32) | ((neg_h & 1) << 12) | ((neg_t & 1) << 13))
    rel_pair = rel_matrices.reshape(REL * 32, 128)
    out = _score(wp, wn, rows2, rel_pair)
    return out[0, 0]


# speedup vs baseline: 1.1603x; 1.0800x over previous
import functools

import jax
import jax.numpy as jnp
from jax import lax
from jax.experimental import pallas as pl
from jax.experimental.pallas import tpu as pltpu
from jax.experimental.pallas import tpu_sc as plsc

ENT = 1000000
REL = 1000
H = 64
B = 4096
MARGIN = 1.0

NC = 2
NS = 16
NW = NC * NS
ROWS_PER_W = 4 * B // NW


def _sc_gather_body(idx_hbm, table_hbm, out_hbm, idx_v, rows_v, sem):
    wid = lax.axis_index("s") * NC + lax.axis_index("c")
    base = wid * ROWS_PER_W
    pltpu.sync_copy(idx_hbm.at[pl.ds(base, ROWS_PER_W)], idx_v)
    pltpu.async_copy(table_hbm.at[idx_v], rows_v, sem).wait()
    pltpu.sync_copy(rows_v, out_hbm.at[pl.ds(base, ROWS_PER_W)])


def _sc_gather(idx, table):
    mesh = plsc.VectorSubcoreMesh(core_axis_name="c", subcore_axis_name="s")
    return pl.kernel(
        _sc_gather_body,
        mesh=mesh,
        out_type=jax.ShapeDtypeStruct((4 * B, H), jnp.float32),
        scratch_types=[
            pltpu.VMEM((ROWS_PER_W,), jnp.int32),
            pltpu.VMEM((ROWS_PER_W, H), jnp.float32),
            pltpu.SemaphoreType.DMA,
        ],
        compiler_params=pltpu.CompilerParams(use_tc_tiling_on_sc=False),
    )(idx, table)


KU = 8


def _deintl_matrix():
    ii = lax.broadcasted_iota(jnp.int32, (H, H), 0)
    jj = lax.broadcasted_iota(jnp.int32, (H, H), 1)
    src = 2 * (jj % (H // 2)) + jnp.where(jj < H // 2, 0, 1)
    return (ii == src).astype(jnp.float32)


def _score_body(rp_ref, rn_ref, rows_ref, relp_ref, out_ref,
                hde_ref, diff_ref):
    P = _deintl_matrix()
    hde_ref[pl.ds(0, B), :] = jnp.dot(
        rows_ref[pl.ds(0, B), :], P, preferred_element_type=jnp.float32)
    hde_ref[pl.ds(B, B), :] = jnp.dot(
        rows_ref[pl.ds(2 * B, B), :], P, preferred_element_type=jnp.float32)

    def body(i, acc):
        b0 = i * KU
        for u in range(KU):
            b = b0 + u
            rp = rp_ref[b]
            rn = rn_ref[b]
            Rp = relp_ref[pl.ds(rp * 32, 32), :]
            Rn = relp_ref[pl.ds(rn * 32, 32), :]
            hp = hde_ref[pl.ds(b, 1), :]
            hn = hde_ref[pl.ds(B + b, 1), :]
            tp = rows_ref[pl.ds(B + b, 1), :]
            tn = rows_ref[pl.ds(3 * B + b, 1), :]
            pe = jnp.dot(lax.slice(hp, (0, 0), (1, 32)), Rp,
                         preferred_element_type=jnp.float32)
            po = jnp.dot(lax.slice(hp, (0, 32), (1, 64)), Rp,
                         preferred_element_type=jnp.float32)
            ne = jnp.dot(lax.slice(hn, (0, 0), (1, 32)), Rn,
                         preferred_element_type=jnp.float32)
            no = jnp.dot(lax.slice(hn, (0, 32), (1, 64)), Rn,
                         preferred_element_type=jnp.float32)
            pvec = (lax.slice(pe, (0, 0), (1, H))
                    + lax.slice(po, (0, H), (1, 2 * H))) * tp
            nvec = (lax.slice(ne, (0, 0), (1, H))
                    + lax.slice(no, (0, H), (1, 2 * H))) * tn
            diff_ref[pl.ds(b, 1), :] = nvec - pvec
        return acc

    lax.fori_loop(0, B // KU, body, jnp.float32(0.0))
    d = diff_ref[...]
    s = jnp.sum(d, axis=1) + MARGIN
    out_ref[0, 0] = jnp.sum(jnp.maximum(s, 0.0))


def _score(rp, rn, rows, rel_pair):
    return pl.pallas_call(
        _score_body,
        out_shape=jax.ShapeDtypeStruct((1, 1), jnp.float32),
        in_specs=[
            pl.BlockSpec(memory_space=pltpu.SMEM),
            pl.BlockSpec(memory_space=pltpu.SMEM),
            pl.BlockSpec(memory_space=pltpu.VMEM),
            pl.BlockSpec(memory_space=pltpu.VMEM),
        ],
        out_specs=pl.BlockSpec(memory_space=pltpu.SMEM),
        scratch_shapes=[
            pltpu.VMEM((2 * B, H), jnp.float32),
            pltpu.VMEM((B, H), jnp.float32),
        ],
    )(rp, rn, rows, rel_pair)


def kernel(pos_h, pos_t, pos_r, neg_h, neg_t, neg_r,
           ent_embeddings, rel_matrices):
    idx = jnp.concatenate([pos_h, pos_t, neg_h, neg_t]).astype(jnp.int32)
    rows = _sc_gather(idx, ent_embeddings)
    rel_pair = rel_matrices.reshape(REL * 32, 128)
    out = _score(pos_r.astype(jnp.int32), neg_r.astype(jnp.int32),
                 rows, rel_pair)
    return out[0, 0]


# speedup vs baseline: 2.1567x; 1.8588x over previous
import functools

import jax
import jax.numpy as jnp
from jax import lax
from jax.experimental import pallas as pl
from jax.experimental.pallas import tpu as pltpu
from jax.experimental.pallas import tpu_sc as plsc

ENT = 1000000
REL = 1000
H = 64
B = 4096
MARGIN = 1.0

NC = 2
NS = 16
NW = NC * NS
ROWS_PER_W = 4 * B // NW


CH = 32
NCHUNK = ROWS_PER_W // CH


def _sc_gather_body(tidx_hbm, sub_hbm, table_hbm, out_hbm,
                    tidx_v, sub_v, tb0, tb1, ob0, ob1,
                    sg0, sg1, so0, so1):
    wid = lax.axis_index("s") * NC + lax.axis_index("c")
    base = wid * ROWS_PER_W
    pltpu.sync_copy(tidx_hbm.at[pl.ds(base, ROWS_PER_W)], tidx_v)
    pltpu.sync_copy(sub_hbm.at[pl.ds(base, ROWS_PER_W)], sub_v)

    tb = [tb0, tb1]
    ob = [ob0, ob1]
    sg = [sg0, sg1]
    so = [so0, so1]

    def start_gather(c):
        return pltpu.async_copy(
            table_hbm.at[tidx_v.at[pl.ds(c * CH, CH)]], tb[c % 2], sg[c % 2])

    def extract(c):
        coff = c * CH
        for half in range(CH // 16):
            jv = lax.iota(jnp.int32, 16) + (half * 16)
            sv = sub_v[pl.ds(coff + half * 16, 16)]

            def col_body(k, _):
                kv = jnp.zeros((16,), jnp.int32) + k
                v = plsc.load_gather(tb[c % 2], [jv, sv, kv])
                plsc.store_scatter(ob[c % 2], [jv, kv], v)
                return 0

            lax.fori_loop(0, H, col_body, 0)

    g = [start_gather(0)]
    w = [None, None]
    for c in range(NCHUNK):
        g[c].wait()
        if c + 1 < NCHUNK:
            g.append(start_gather(c + 1))
        if c >= 2:
            w[c % 2].wait()
        extract(c)
        w[c % 2] = pltpu.async_copy(
            ob[c % 2], out_hbm.at[pl.ds(base + c * CH, CH)], so[c % 2])
    w[0].wait()
    w[1].wait()


def _sc_gather(idx, table):
    mesh = plsc.VectorSubcoreMesh(core_axis_name="c", subcore_axis_name="s")
    table3 = table.reshape(ENT // 8, 8, H)
    return pl.kernel(
        _sc_gather_body,
        mesh=mesh,
        out_type=jax.ShapeDtypeStruct((4 * B, H), jnp.float32),
        scratch_types=[
            pltpu.VMEM((ROWS_PER_W,), jnp.int32),
            pltpu.VMEM((ROWS_PER_W,), jnp.int32),
            pltpu.VMEM((CH, 8, H), jnp.float32),
            pltpu.VMEM((CH, 8, H), jnp.float32),
            pltpu.VMEM((CH, H), jnp.float32),
            pltpu.VMEM((CH, H), jnp.float32),
            pltpu.SemaphoreType.DMA,
            pltpu.SemaphoreType.DMA,
            pltpu.SemaphoreType.DMA,
            pltpu.SemaphoreType.DMA,
        ],
        compiler_params=pltpu.CompilerParams(needs_layout_passes=False),
    )(idx >> 3, idx & 7, table3)


KU = 8


def _deintl_matrix():
    ii = lax.broadcasted_iota(jnp.int32, (H, H), 0)
    jj = lax.broadcasted_iota(jnp.int32, (H, H), 1)
    src = 2 * (jj % (H // 2)) + jnp.where(jj < H // 2, 0, 1)
    return (ii == src).astype(jnp.float32)


def _score_body(rp_ref, rn_ref, rows_ref, relp_ref, out_ref,
                hde_ref, diff_ref):
    P = _deintl_matrix()
    hde_ref[pl.ds(0, B), :] = jnp.dot(
        rows_ref[pl.ds(0, B), :], P, preferred_element_type=jnp.float32)
    hde_ref[pl.ds(B, B), :] = jnp.dot(
        rows_ref[pl.ds(2 * B, B), :], P, preferred_element_type=jnp.float32)

    def body(i, acc):
        b0 = i * KU
        for u in range(KU):
            b = b0 + u
            rp = rp_ref[b]
            rn = rn_ref[b]
            Rp = relp_ref[pl.ds(rp * 32, 32), :]
            Rn = relp_ref[pl.ds(rn * 32, 32), :]
            hp = hde_ref[pl.ds(b, 1), :]
            hn = hde_ref[pl.ds(B + b, 1), :]
            tp = rows_ref[pl.ds(B + b, 1), :]
            tn = rows_ref[pl.ds(3 * B + b, 1), :]
            pe = jnp.dot(lax.slice(hp, (0, 0), (1, 32)), Rp,
                         preferred_element_type=jnp.float32)
            po = jnp.dot(lax.slice(hp, (0, 32), (1, 64)), Rp,
                         preferred_element_type=jnp.float32)
            ne = jnp.dot(lax.slice(hn, (0, 0), (1, 32)), Rn,
                         preferred_element_type=jnp.float32)
            no = jnp.dot(lax.slice(hn, (0, 32), (1, 64)), Rn,
                         preferred_element_type=jnp.float32)
            pvec = (lax.slice(pe, (0, 0), (1, H))
                    + lax.slice(po, (0, H), (1, 2 * H))) * tp
            nvec = (lax.slice(ne, (0, 0), (1, H))
                    + lax.slice(no, (0, H), (1, 2 * H))) * tn
            diff_ref[pl.ds(b, 1), :] = nvec - pvec
        return acc

    lax.fori_loop(0, B // KU, body, jnp.float32(0.0))
    d = diff_ref[...]
    s = jnp.sum(d, axis=1) + MARGIN
    out_ref[0, 0] = jnp.sum(jnp.maximum(s, 0.0))


def _score(rp, rn, rows, rel_pair):
    return pl.pallas_call(
        _score_body,
        out_shape=jax.ShapeDtypeStruct((1, 1), jnp.float32),
        in_specs=[
            pl.BlockSpec(memory_space=pltpu.SMEM),
            pl.BlockSpec(memory_space=pltpu.SMEM),
            pl.BlockSpec(memory_space=pltpu.VMEM),
            pl.BlockSpec(memory_space=pltpu.VMEM),
        ],
        out_specs=pl.BlockSpec(memory_space=pltpu.SMEM),
        scratch_shapes=[
            pltpu.VMEM((2 * B, H), jnp.float32),
            pltpu.VMEM((B, H), jnp.float32),
        ],
    )(rp, rn, rows, rel_pair)


def kernel(pos_h, pos_t, pos_r, neg_h, neg_t, neg_r,
           ent_embeddings, rel_matrices):
    idx = jnp.concatenate([pos_h, pos_t, neg_h, neg_t]).astype(jnp.int32)
    rows = jnp.take(ent_embeddings, idx, axis=0)
    rel_pair = rel_matrices.reshape(REL * 32, 128)
    out = _score(pos_r.astype(jnp.int32), neg_r.astype(jnp.int32),
                 rows, rel_pair)
    return out[0, 0]
